# baseline (speedup 1.0000x reference)
import jax
import jax.numpy as jnp
import numpy as np
from jax.experimental import pallas as pl
from jax.experimental.pallas import tpu as pltpu

_H = 128
_W = 128
_K = 8
_BIG = np.float32(1e9)


def _loss_kernel(pred_ref, targ_ref, out_ref, x_ref, g_ref, t_ref, d_ref):
    thr = jnp.float32(0.3) + jnp.float32(1e-5) * jnp.float32(1.0)

    def masks(k):
        pm = jnp.abs(pred_ref[k] - jnp.float32(1.0)) <= thr
        tg = targ_ref[k] != jnp.float32(0.0)
        return pm, tg

    for k in range(_K):
        pm, tg = masks(k)
        x_ref[:, k * _W:(k + 1) * _W] = jnp.where(
            pm, jnp.float32(0.0), _BIG).T
        x_ref[:, (_K + k) * _W:(_K + k + 1) * _W] = jnp.where(
            tg, jnp.float32(0.0), _BIG).T

    g_ref[0:1, :] = x_ref[0:1, :]

    def fwd(u, g):
        r = 2 * u + 1
        g = jnp.minimum(x_ref[pl.ds(r, 1), :], g + jnp.float32(1.0))
        g_ref[pl.ds(r, 1), :] = g
        g = jnp.minimum(x_ref[pl.ds(r + 1, 1), :], g + jnp.float32(1.0))
        g_ref[pl.ds(r + 1, 1), :] = g
        return g

    gl = jax.lax.fori_loop(0, (_H - 2) // 2, fwd, x_ref[0:1, :])
    b0 = jnp.minimum(x_ref[_H - 1:_H, :], gl + jnp.float32(1.0))
    g_ref[_H - 1:_H, :] = b0 * b0

    def bwd(u, b):
        r = _H - 2 - 2 * u
        b = jnp.minimum(g_ref[pl.ds(r, 1), :], b + jnp.float32(1.0))
        g_ref[pl.ds(r, 1), :] = b * b
        b2 = jnp.minimum(g_ref[pl.ds(r - 1, 1), :], b + jnp.float32(1.0))
        g_ref[pl.ds(r - 1, 1), :] = b2 * b2
        return b2

    blast = jax.lax.fori_loop(0, (_H - 2) // 2, bwd, b0)
    bfin = jnp.minimum(g_ref[0:1, :], blast + jnp.float32(1.0))
    g_ref[0:1, :] = bfin * bfin

    for k in range(2 * _K):
        blk = g_ref[:, k * _W:(k + 1) * _W]
        t_ref[:, k * _W:(k + 1) * _W] = blk.T

    o_col = jax.lax.broadcasted_iota(jnp.int32, (_H, 1), 0).astype(jnp.float32)

    _UNROLL = 8

    def candn(s_f32, base):
        c = None
        for q in range(_UNROLL):
            d = o_col - (s_f32 + jnp.float32(q))
            term = t_ref[pl.ds(base + q, 1), :] + d * d
            c = term if c is None else jnp.minimum(c, term)
        return c

    d_ref[...] = candn(jnp.float32(0.0), 0)

    def p2(u, carry):
        s = _UNROLL * u
        d_ref[...] = jnp.minimum(d_ref[...], candn(s.astype(jnp.float32), s))
        return carry

    jax.lax.fori_loop(1, _H // _UNROLL, p2, jnp.int32(0))

    total = jnp.float32(0.0)
    for k in range(_K):
        pm, tg = masks(k)
        d2_pred = d_ref[:, k * _W:(k + 1) * _W]
        d2_tgt = d_ref[:, (_K + k) * _W:(_K + k + 1) * _W]
        n_pred = jnp.sum(pm.astype(jnp.float32))
        n_gt = jnp.sum(tg.astype(jnp.float32))
        s1 = jnp.sum(jnp.where(pm, jnp.sqrt(d2_tgt), jnp.float32(0.0)))
        s2 = jnp.sum(jnp.where(tg, jnp.sqrt(d2_pred), jnp.float32(0.0)))
        term = (s1 + s2) / (jnp.float32(2.0) * n_gt)
        term = jnp.where((n_pred == 0.0) | (n_gt == 0.0), jnp.float32(0.0),
                         term)
        total = total + term
    out_ref[0, 0] = total / jnp.float32(_K)


def kernel(pred, target):
    n = pred.shape[0] * pred.shape[1]
    pred3 = pred.reshape(n, _H, _W)
    targ3 = target.reshape(n, _H, _W)
    out = pl.pallas_call(
        _loss_kernel,
        out_shape=jax.ShapeDtypeStruct((1, 1), jnp.float32),
        in_specs=[pl.BlockSpec(memory_space=pltpu.VMEM)] * 2,
        out_specs=pl.BlockSpec(memory_space=pltpu.SMEM),
        scratch_shapes=[
            pltpu.VMEM((_H, 2 * _K * _W), jnp.float32),
            pltpu.VMEM((_H, 2 * _K * _W), jnp.float32),
            pltpu.VMEM((_H, 2 * _K * _W), jnp.float32),
            pltpu.VMEM((_H, 2 * _K * _W), jnp.float32),
        ],
    )(pred3, targ3)
    return out[0, 0]


# speedup vs baseline: 1.0028x; 1.0028x over previous
import jax
import jax.numpy as jnp
import numpy as np
from jax.experimental import pallas as pl
from jax.experimental.pallas import tpu as pltpu

_H = 128
_W = 128
_K = 8
_BIG = np.float32(1e9)


def _loss_kernel(pred_ref, targ_ref, out_ref, x_ref, g_ref, t_ref, d_ref):
    thr = jnp.float32(0.3) + jnp.float32(1e-5) * jnp.float32(1.0)

    def masks(k):
        pm = jnp.abs(pred_ref[k] - jnp.float32(1.0)) <= thr
        tg = targ_ref[k] != jnp.float32(0.0)
        return pm, tg

    for k in range(_K):
        pm, tg = masks(k)
        x_ref[:, k * _W:(k + 1) * _W] = jnp.where(
            pm, jnp.float32(0.0), _BIG).T
        x_ref[:, (_K + k) * _W:(_K + k + 1) * _W] = jnp.where(
            tg, jnp.float32(0.0), _BIG).T

    g_ref[0:1, :] = x_ref[0:1, :]

    def fwd(u, g):
        r = 2 * u + 1
        g = jnp.minimum(x_ref[pl.ds(r, 1), :], g + jnp.float32(1.0))
        g_ref[pl.ds(r, 1), :] = g
        g = jnp.minimum(x_ref[pl.ds(r + 1, 1), :], g + jnp.float32(1.0))
        g_ref[pl.ds(r + 1, 1), :] = g
        return g

    gl = jax.lax.fori_loop(0, (_H - 2) // 2, fwd, x_ref[0:1, :])
    b0 = jnp.minimum(x_ref[_H - 1:_H, :], gl + jnp.float32(1.0))
    g_ref[_H - 1:_H, :] = b0 * b0

    def bwd(u, b):
        r = _H - 2 - 2 * u
        b = jnp.minimum(g_ref[pl.ds(r, 1), :], b + jnp.float32(1.0))
        g_ref[pl.ds(r, 1), :] = b * b
        b2 = jnp.minimum(g_ref[pl.ds(r - 1, 1), :], b + jnp.float32(1.0))
        g_ref[pl.ds(r - 1, 1), :] = b2 * b2
        return b2

    blast = jax.lax.fori_loop(0, (_H - 2) // 2, bwd, b0)
    bfin = jnp.minimum(g_ref[0:1, :], blast + jnp.float32(1.0))
    g_ref[0:1, :] = bfin * bfin

    for k in range(2 * _K):
        blk = g_ref[:, k * _W:(k + 1) * _W]
        t_ref[:, k * _W:(k + 1) * _W] = blk.T

    o_col = jax.lax.broadcasted_iota(jnp.int32, (_H, 1), 0).astype(jnp.float32)

    def cand4(s_f32, base):
        c = None
        for q in range(4):
            d = o_col - (s_f32 + jnp.float32(q))
            term = t_ref[pl.ds(base + q, 1), :] + d * d
            c = term if c is None else jnp.minimum(c, term)
        return c

    d_ref[...] = cand4(jnp.float32(0.0), 0)

    def p2(u, carry):
        s = 4 * u
        d_ref[...] = jnp.minimum(d_ref[...], cand4(s.astype(jnp.float32), s))
        return carry

    jax.lax.fori_loop(1, _H // 4, p2, jnp.int32(0))

    total = jnp.float32(0.0)
    for k in range(_K):
        pm, tg = masks(k)
        d2_pred = d_ref[:, k * _W:(k + 1) * _W]
        d2_tgt = d_ref[:, (_K + k) * _W:(_K + k + 1) * _W]
        n_pred = jnp.sum(pm.astype(jnp.float32))
        n_gt = jnp.sum(tg.astype(jnp.float32))
        s1 = jnp.sum(jnp.where(pm, jnp.sqrt(d2_tgt), jnp.float32(0.0)))
        s2 = jnp.sum(jnp.where(tg, jnp.sqrt(d2_pred), jnp.float32(0.0)))
        term = (s1 + s2) / (jnp.float32(2.0) * n_gt)
        term = jnp.where((n_pred == 0.0) | (n_gt == 0.0), jnp.float32(0.0),
                         term)
        total = total + term
    out_ref[0, 0] = total / jnp.float32(_K)


def kernel(pred, target):
    n = pred.shape[0] * pred.shape[1]
    pred3 = pred.reshape(n, _H, _W)
    targ3 = target.reshape(n, _H, _W)
    out = pl.pallas_call(
        _loss_kernel,
        out_shape=jax.ShapeDtypeStruct((1, 1), jnp.float32),
        in_specs=[pl.BlockSpec(memory_space=pltpu.VMEM)] * 2,
        out_specs=pl.BlockSpec(memory_space=pltpu.SMEM),
        scratch_shapes=[
            pltpu.VMEM((_H, 2 * _K * _W), jnp.float32),
            pltpu.VMEM((_H, 2 * _K * _W), jnp.float32),
            pltpu.VMEM((_H, 2 * _K * _W), jnp.float32),
            pltpu.VMEM((_H, 2 * _K * _W), jnp.float32),
        ],
    )(pred3, targ3)
    return out[0, 0]
